# Initial kernel scaffold; baseline (speedup 1.0000x reference)
import jax
import jax.numpy as jnp
from jax.experimental import pallas as pl


def kernel(node_invariant, node_equivariant, rbf, fcut, rsh, edge_index, W_q, W_k, W_v, W_qs, W_ks, W_vs, rbf_w1, rbf_b1, rbf_w2, rbf_b2, inv_w1, inv_b1, inv_w2, inv_b2):
    raise NotImplementedError("write your pallas kernel here")


# speedup vs baseline: 3.9211x; 3.9211x over previous
import functools
import math

import jax
import jax.numpy as jnp
import numpy as np
from jax import lax
from jax.experimental import pallas as pl
from jax.experimental.pallas import tpu as pltpu
from jax.experimental.pallas import tpu_sc as plsc

N = 10000
E = 160000
D = 128
MUL = 32
NB = 20
CTR_W = 384
NBR_W = 640

NC = 2
NS = 16
CH = 128
NCHUNK = E // CH


def _const_mats():
    S = np.zeros((D, 64), np.float32)
    for m in range(MUL):
        S[m, m] = 1.0
        for t in range(3):
            S[MUL + 3 * m + t, MUL + m] = 1.0
    B32 = np.zeros((D, D), np.float32)
    for i in range(D):
        for j in range(D):
            if i // 32 == j // 32:
                B32[i, j] = 1.0
    P = np.zeros((D, 64), np.float32)
    for i in range(D):
        for j in range(64):
            if i // 64 == j // 32:
                P[i, j] = 1.0
    R = np.zeros((64, D), np.float32)
    for k in range(D):
        R[k if k < MUL else MUL + (k - MUL) // 3, k] = 1.0
    return S, B32, P, R


_S, _B32, _P, _R = _const_mats()


def _proj_body(x_ref, wq_ref, wqs_ref, wk_ref, wv_ref, wks_ref, wvs_ref,
               ctr_ref, nbr_ref):
    x = x_ref[...]
    dn = (((1,), (1,)), ((), ()))
    dot = functools.partial(lax.dot_general, dimension_numbers=dn,
                            preferred_element_type=jnp.float32)
    ctr_ref[:, 0:D] = x
    ctr_ref[:, D:2 * D] = dot(x, wq_ref[...])
    ctr_ref[:, 2 * D:3 * D] = dot(x, wqs_ref[...])
    nbr_ref[:, 0:D] = x
    nbr_ref[:, D:2 * D] = dot(x, wk_ref[...])
    nbr_ref[:, 2 * D:3 * D] = dot(x, wv_ref[...])
    nbr_ref[:, 3 * D:4 * D] = dot(x, wks_ref[...])
    vs = dot(x, wvs_ref[...])
    nbr_ref[:, 4 * D:4 * D + 64] = vs
    nbr_ref[:, 4 * D + 64:5 * D] = jnp.zeros_like(vs)


def _proj(x, wq, wqs, wk, wv, wks, wvs, *, interpret=False):
    bn = 1000
    grid = (N // bn,)
    full = lambda a: pl.BlockSpec(a.shape, lambda i: (0,) * a.ndim)
    return pl.pallas_call(
        _proj_body,
        grid=grid,
        in_specs=[pl.BlockSpec((bn, D), lambda i: (i, 0))] +
                 [full(w) for w in (wq, wqs, wk, wv, wks, wvs)],
        out_specs=[pl.BlockSpec((bn, CTR_W), lambda i: (i, 0)),
                   pl.BlockSpec((bn, NBR_W), lambda i: (i, 0))],
        out_shape=[jax.ShapeDtypeStruct((N, CTR_W), jnp.float32),
                   jax.ShapeDtypeStruct((N, NBR_W), jnp.float32)],
        interpret=interpret,
    )(x, wq, wqs, wk, wv, wks, wvs)


def _gather_body(nchunk, ni, table_ref, idx_hbm, out_ref, idx_v, rows_v, sem):
    wid = lax.axis_index("c") * NS + lax.axis_index("s")

    def step(i, carry):
        c = wid + NC * NS * i

        @pl.when(c < nchunk)
        def _():
            base = c * CH
            pltpu.sync_copy(idx_hbm.at[pl.ds(base, CH)], idx_v)
            pltpu.async_copy(table_ref.at[idx_v], rows_v, sem).wait()
            pltpu.sync_copy(rows_v, out_ref.at[pl.ds(base, CH)])

        return carry

    lax.fori_loop(0, ni, step, 0)


def _gather(table, idx, width):
    mesh = plsc.VectorSubcoreMesh(core_axis_name="c", subcore_axis_name="s")
    ni = (NCHUNK + NC * NS - 1) // (NC * NS)
    return pl.kernel(
        functools.partial(_gather_body, NCHUNK, ni),
        out_type=jax.ShapeDtypeStruct((E, width), jnp.float32),
        mesh=mesh,
        scratch_types=[
            pltpu.VMEM((CH,), jnp.int32),
            pltpu.VMEM((CH, width), jnp.float32),
            pltpu.SemaphoreType.DMA,
        ],
    )(table, idx)


def _edge_body(ctr_ref, nbr_ref, rbf_ref, fcut_ref, rsh_ref,
               iw1_ref, ib1_ref, iw2_ref, ib2_ref,
               rw1_ref, rb1_ref, rw2_ref, rb2_ref,
               s_ref, b32_ref, p_ref, r_ref,
               msgs_ref, msge_ref):
    dn = (((1,), (1,)), ((), ()))
    dotT = functools.partial(lax.dot_general, dimension_numbers=dn,
                             preferred_element_type=jnp.float32)
    dot = functools.partial(lax.dot_general,
                            dimension_numbers=(((1,), (0,)), ((), ())),
                            preferred_element_type=jnp.float32)
    ctr = ctr_ref[...]
    nbr = nbr_ref[...]
    fcut = fcut_ref[...]
    x_ij = nbr[:, 0:D] - ctr[:, 0:D]
    inv_x = dot(x_ij * x_ij, s_ref[...])
    h = dotT(inv_x, iw1_ref[...]) + ib1_ref[...]
    h = h * jax.nn.sigmoid(h)
    w_l = dotT(h, iw2_ref[...]) + ib2_ref[...]
    g = dotT(rbf_ref[...], rw1_ref[...]) + rb1_ref[...]
    g = g * jax.nn.sigmoid(g)
    w_r = dotT(g, rw2_ref[...]) + rb2_ref[...]
    w_ij = (w_l + w_r) * fcut
    t = (ctr[:, D:2 * D] * w_ij) * nbr[:, D:2 * D]
    attn_bc = dot(t, b32_ref[...]) * (1.0 / math.sqrt(D))
    msgs_ref[...] = attn_bc * nbr[:, 2 * D:3 * D]
    t2 = (ctr[:, 2 * D:3 * D] * w_ij) * nbr[:, 3 * D:4 * D]
    gate = dot(t2, p_ref[...]) * (1.0 / math.sqrt(64)) * nbr[:, 4 * D:4 * D + 64]
    msge_ref[...] = dot(gate, r_ref[...]) * rsh_ref[...] * fcut


def _edge(ctr_rows, nbr_rows, rbf, fcut, rsh, weights, *, interpret=False):
    be = 1000
    grid = (E // be,)
    full = lambda a: pl.BlockSpec(a.shape, lambda i: (0,) * a.ndim)
    row = lambda w: pl.BlockSpec((be, w), lambda i: (i, 0))
    return pl.pallas_call(
        _edge_body,
        grid=grid,
        in_specs=[row(CTR_W), row(NBR_W), row(NB), row(1), row(D)] +
                 [full(w) for w in weights],
        out_specs=[row(D), row(D)],
        out_shape=[jax.ShapeDtypeStruct((E, D), jnp.float32),
                   jax.ShapeDtypeStruct((E, D), jnp.float32)],
        interpret=interpret,
    )(ctr_rows, nbr_rows, rbf, fcut, rsh, *weights)


def _scatter_body(msgs_hbm, msge_hbm, bases_hbm, basee_hbm, idx_hbm,
                  outs_ref, oute_ref, msg_v, idx_v, acc):
    cid = lax.axis_index("c")
    sid = lax.axis_index("s")
    rows = 632
    last_r0 = (NS - 1) * rows
    last_rows = N - last_r0
    ni = (NCHUNK + NS - 1) // NS

    def run(msg_hbm, base_hbm, out_ref):
        def slab_copy(src, dst):
            @pl.when(sid < NS - 1)
            def _():
                s = pl.ds(sid * rows, rows)
                pltpu.sync_copy(src.at[s], dst.at[s])

            @pl.when(sid == NS - 1)
            def _():
                s = pl.ds(last_r0, last_rows)
                pltpu.sync_copy(src.at[s], dst.at[s])

        slab_copy(base_hbm, acc)
        plsc.subcore_barrier()

        def step(i, carry):
            c = sid + NS * i

            @pl.when(c < NCHUNK)
            def _():
                base = c * CH
                pltpu.sync_copy(idx_hbm.at[pl.ds(base, CH)], idx_v)
                pltpu.sync_copy(msg_hbm.at[pl.ds(base, CH)], msg_v)
                pltpu.sync_copy(msg_v, acc.at[idx_v], add=True)

            return carry

        lax.fori_loop(0, ni, step, 0)
        plsc.subcore_barrier()
        slab_copy(acc, out_ref)

    @pl.when(cid == 0)
    def _():
        run(msgs_hbm, bases_hbm, outs_ref)

    @pl.when(cid == 1)
    def _():
        run(msge_hbm, basee_hbm, oute_ref)


def _scatter(msg_s, msg_e, base_s, base_e, idx):
    mesh = plsc.VectorSubcoreMesh(core_axis_name="c", subcore_axis_name="s")
    return pl.kernel(
        _scatter_body,
        out_type=[jax.ShapeDtypeStruct((N, D), jnp.float32),
                  jax.ShapeDtypeStruct((N, D), jnp.float32)],
        mesh=mesh,
        scratch_types=[
            pltpu.VMEM((CH, D), jnp.float32),
            pltpu.VMEM((CH,), jnp.int32),
            pltpu.VMEM_SHARED((N, D), jnp.float32),
        ],
    )(msg_s, msg_e, base_s, base_e, idx)


def kernel(node_invariant, node_equivariant, rbf, fcut, rsh, edge_index,
           W_q, W_k, W_v, W_qs, W_ks, W_vs,
           rbf_w1, rbf_b1, rbf_w2, rbf_b2,
           inv_w1, inv_b1, inv_w2, inv_b2):
    center = edge_index[0]
    neighbor = edge_index[1]
    ctr_tab, nbr_tab = _proj(node_invariant, W_q, W_qs, W_k, W_v, W_ks, W_vs)
    ctr_rows = _gather(ctr_tab, center, CTR_W)
    nbr_rows = _gather(nbr_tab, neighbor, NBR_W)
    weights = (inv_w1, inv_b1.reshape(1, D), inv_w2, inv_b2.reshape(1, D),
               rbf_w1, rbf_b1.reshape(1, D), rbf_w2, rbf_b2.reshape(1, D),
               jnp.asarray(_S), jnp.asarray(_B32), jnp.asarray(_P),
               jnp.asarray(_R))
    msg_s, msg_e = _edge(ctr_rows, nbr_rows, rbf, fcut, rsh, weights)
    out_s, out_e = _scatter(msg_s, msg_e, node_invariant, node_equivariant, center)
    return out_s, out_e


# speedup vs baseline: 4.2759x; 1.0905x over previous
import functools
import math

import jax
import jax.numpy as jnp
import numpy as np
from jax import lax
from jax.experimental import pallas as pl
from jax.experimental.pallas import tpu as pltpu
from jax.experimental.pallas import tpu_sc as plsc

N = 10000
E = 160000
D = 128
MUL = 32
NB = 20
CTR_W = 384
NBR_W = 640

NC = 2
NS = 16
CH = 128
NCHUNK = E // CH


def _const_mats():
    S = np.zeros((D, 64), np.float32)
    for m in range(MUL):
        S[m, m] = 1.0
        for t in range(3):
            S[MUL + 3 * m + t, MUL + m] = 1.0
    B32 = np.zeros((D, D), np.float32)
    for i in range(D):
        for j in range(D):
            if i // 32 == j // 32:
                B32[i, j] = 1.0
    P = np.zeros((D, 64), np.float32)
    for i in range(D):
        for j in range(64):
            if i // 64 == j // 32:
                P[i, j] = 1.0
    R = np.zeros((64, D), np.float32)
    for k in range(D):
        R[k if k < MUL else MUL + (k - MUL) // 3, k] = 1.0
    return S, B32, P, R


_S, _B32, _P, _R = _const_mats()


def _proj_body(x_ref, wq_ref, wqs_ref, wk_ref, wv_ref, wks_ref, wvs_ref,
               ctr_ref, nbr_ref):
    x = x_ref[...]
    dn = (((1,), (1,)), ((), ()))
    dot = functools.partial(lax.dot_general, dimension_numbers=dn,
                            preferred_element_type=jnp.float32)
    ctr_ref[:, 0:D] = x
    ctr_ref[:, D:2 * D] = dot(x, wq_ref[...])
    ctr_ref[:, 2 * D:3 * D] = dot(x, wqs_ref[...])
    nbr_ref[:, 0:D] = x
    nbr_ref[:, D:2 * D] = dot(x, wk_ref[...])
    nbr_ref[:, 2 * D:3 * D] = dot(x, wv_ref[...])
    nbr_ref[:, 3 * D:4 * D] = dot(x, wks_ref[...])
    vs = dot(x, wvs_ref[...])
    nbr_ref[:, 4 * D:4 * D + 64] = vs
    nbr_ref[:, 4 * D + 64:5 * D] = jnp.zeros_like(vs)


def _proj(x, wq, wqs, wk, wv, wks, wvs, *, interpret=False):
    bn = 1000
    grid = (N // bn,)
    full = lambda a: pl.BlockSpec(a.shape, lambda i: (0,) * a.ndim)
    return pl.pallas_call(
        _proj_body,
        grid=grid,
        in_specs=[pl.BlockSpec((bn, D), lambda i: (i, 0))] +
                 [full(w) for w in (wq, wqs, wk, wv, wks, wvs)],
        out_specs=[pl.BlockSpec((bn, CTR_W), lambda i: (i, 0)),
                   pl.BlockSpec((bn, NBR_W), lambda i: (i, 0))],
        out_shape=[jax.ShapeDtypeStruct((N, CTR_W), jnp.float32),
                   jax.ShapeDtypeStruct((N, NBR_W), jnp.float32)],
        interpret=interpret,
    )(x, wq, wqs, wk, wv, wks, wvs)


def _gather_body(nblocks, pw, ch, table_ref, idx2d_hbm, out_ref,
                 idxb, rows0, rows1, g0, g1, w0, w1):
    nw = NC * NS
    wid = lax.axis_index("c") * NS + lax.axis_index("s")
    c0 = wid * pw
    last_cnt = nblocks - (nw - 1) * pw
    cnt = jnp.where(wid < nw - 1, pw, last_cnt)

    pltpu.sync_copy(idx2d_hbm.at[pl.ds(c0, pw)], idxb)

    rows = (rows0, rows1)
    gs = (g0, g1)
    ws = (w0, w1)

    def step(i2, carry):
        for b in range(2):
            i = 2 * i2 + b

            @pl.when(i < cnt)
            def _(b=b, i=i):
                @pl.when(i2 > 0)
                def _():
                    pltpu.make_async_copy(
                        rows[b], out_ref.at[pl.ds(0, ch)], ws[b]).wait()

                pltpu.async_copy(table_ref.at[idxb.at[i]], rows[b], gs[b])

        for b in range(2):
            i = 2 * i2 + b

            @pl.when(i < cnt)
            def _(b=b, i=i):
                pltpu.make_async_copy(
                    table_ref.at[idxb.at[i]], rows[b], gs[b]).wait()
                pltpu.async_copy(rows[b], out_ref.at[pl.ds((c0 + i) * ch, ch)],
                                 ws[b])

        return carry

    lax.fori_loop(0, pw // 2, step, 0)

    for b in range(2):
        @pl.when(b < cnt)
        def _(b=b):
            pltpu.make_async_copy(rows[b], out_ref.at[pl.ds(0, ch)],
                                  ws[b]).wait()


def _gather(table, idx, width, ch):
    mesh = plsc.VectorSubcoreMesh(core_axis_name="c", subcore_axis_name="s")
    nblocks = E // ch
    pw = -(-nblocks // (NC * NS))
    pw += pw % 2
    call = pl.kernel(
        functools.partial(_gather_body, nblocks, pw, ch),
        out_type=jax.ShapeDtypeStruct((E, width), jnp.float32),
        mesh=mesh,
        scratch_types=[
            pltpu.VMEM((pw, ch), jnp.int32),
            pltpu.VMEM((ch, width), jnp.float32),
            pltpu.VMEM((ch, width), jnp.float32),
            pltpu.SemaphoreType.DMA,
            pltpu.SemaphoreType.DMA,
            pltpu.SemaphoreType.DMA,
            pltpu.SemaphoreType.DMA,
        ],
    )
    idx2d = jnp.pad(idx.reshape(nblocks, ch),
                    ((0, NC * NS * pw - nblocks), (0, 0)))
    return call(table, idx2d)


def _edge_body(ctr_ref, nbr_ref, rbf_ref, fcut_ref, rsh_ref,
               iw1_ref, ib1_ref, iw2_ref, ib2_ref,
               rw1_ref, rb1_ref, rw2_ref, rb2_ref,
               s_ref, b32_ref, p_ref, r_ref,
               msgs_ref, msge_ref):
    dn = (((1,), (1,)), ((), ()))
    dotT = functools.partial(lax.dot_general, dimension_numbers=dn,
                             preferred_element_type=jnp.float32)
    dot = functools.partial(lax.dot_general,
                            dimension_numbers=(((1,), (0,)), ((), ())),
                            preferred_element_type=jnp.float32)
    ctr = ctr_ref[...]
    nbr = nbr_ref[...]
    fcut = fcut_ref[...]
    x_ij = nbr[:, 0:D] - ctr[:, 0:D]
    inv_x = dot(x_ij * x_ij, s_ref[...])
    h = dotT(inv_x, iw1_ref[...]) + ib1_ref[...]
    h = h * jax.nn.sigmoid(h)
    w_l = dotT(h, iw2_ref[...]) + ib2_ref[...]
    g = dotT(rbf_ref[...], rw1_ref[...]) + rb1_ref[...]
    g = g * jax.nn.sigmoid(g)
    w_r = dotT(g, rw2_ref[...]) + rb2_ref[...]
    w_ij = (w_l + w_r) * fcut
    t = (ctr[:, D:2 * D] * w_ij) * nbr[:, D:2 * D]
    attn_bc = dot(t, b32_ref[...]) * (1.0 / math.sqrt(D))
    msgs_ref[...] = attn_bc * nbr[:, 2 * D:3 * D]
    t2 = (ctr[:, 2 * D:3 * D] * w_ij) * nbr[:, 3 * D:4 * D]
    gate = dot(t2, p_ref[...]) * (1.0 / math.sqrt(64)) * nbr[:, 4 * D:4 * D + 64]
    msge_ref[...] = dot(gate, r_ref[...]) * rsh_ref[...] * fcut


def _edge(ctr_rows, nbr_rows, rbf, fcut, rsh, weights, *, interpret=False):
    be = 1000
    grid = (E // be,)
    full = lambda a: pl.BlockSpec(a.shape, lambda i: (0,) * a.ndim)
    row = lambda w: pl.BlockSpec((be, w), lambda i: (i, 0))
    return pl.pallas_call(
        _edge_body,
        grid=grid,
        in_specs=[row(CTR_W), row(NBR_W), row(NB), row(1), row(D)] +
                 [full(w) for w in weights],
        out_specs=[row(D), row(D)],
        out_shape=[jax.ShapeDtypeStruct((E, D), jnp.float32),
                   jax.ShapeDtypeStruct((E, D), jnp.float32)],
        interpret=interpret,
    )(ctr_rows, nbr_rows, rbf, fcut, rsh, *weights)


def _scatter_body(nblocks, pt, msgs_hbm, msge_hbm, bases_hbm, basee_hbm,
                  idx2d_hbm, outs_ref, oute_ref, msg0, msg1, idxb, acc,
                  m0, m1, s0, s1):
    cid = lax.axis_index("c")
    sid = lax.axis_index("s")
    rows = 632
    last_r0 = (NS - 1) * rows
    last_rows = N - last_r0
    c0 = sid * pt
    last_cnt = nblocks - (NS - 1) * pt
    cnt = jnp.where(sid < NS - 1, pt, last_cnt)
    msg = (msg0, msg1)
    ms = (m0, m1)
    ss = (s0, s1)

    def run(msg_hbm, base_hbm, out_ref):
        def slab_copy(src, dst):
            @pl.when(sid < NS - 1)
            def _():
                s = pl.ds(sid * rows, rows)
                pltpu.sync_copy(src.at[s], dst.at[s])

            @pl.when(sid == NS - 1)
            def _():
                s = pl.ds(last_r0, last_rows)
                pltpu.sync_copy(src.at[s], dst.at[s])

        pltpu.sync_copy(idx2d_hbm.at[pl.ds(c0, pt)], idxb)

        slab_copy(base_hbm, acc)
        plsc.subcore_barrier()

        def step(i2, carry):
            for b in range(2):
                i = 2 * i2 + b

                @pl.when(i < cnt)
                def _(b=b, i=i):
                    @pl.when(i2 > 0)
                    def _():
                        pltpu.make_async_copy(
                            msg[b], acc.at[idxb.at[0]], ss[b]).wait()

                    pltpu.async_copy(msg_hbm.at[pl.ds((c0 + i) * CH, CH)],
                                     msg[b], ms[b])

            for b in range(2):
                i = 2 * i2 + b

                @pl.when(i < cnt)
                def _(b=b, i=i):
                    pltpu.make_async_copy(
                        msg_hbm.at[pl.ds(0, CH)], msg[b], ms[b]).wait()
                    pltpu.async_copy(msg[b], acc.at[idxb.at[i]], ss[b],
                                     add=True)

            return carry

        lax.fori_loop(0, pt // 2, step, 0)

        for b in range(2):
            @pl.when(b < cnt)
            def _(b=b):
                pltpu.make_async_copy(msg[b], acc.at[idxb.at[0]], ss[b]).wait()

        plsc.subcore_barrier()
        slab_copy(acc, out_ref)

    @pl.when(cid == 0)
    def _():
        run(msgs_hbm, bases_hbm, outs_ref)

    @pl.when(cid == 1)
    def _():
        run(msge_hbm, basee_hbm, oute_ref)


def _scatter(msg_s, msg_e, base_s, base_e, idx):
    mesh = plsc.VectorSubcoreMesh(core_axis_name="c", subcore_axis_name="s")
    nblocks = E // CH
    pt = -(-nblocks // NS)
    pt += pt % 2
    call = pl.kernel(
        functools.partial(_scatter_body, nblocks, pt),
        out_type=[jax.ShapeDtypeStruct((N, D), jnp.float32),
                  jax.ShapeDtypeStruct((N, D), jnp.float32)],
        mesh=mesh,
        scratch_types=[
            pltpu.VMEM((CH, D), jnp.float32),
            pltpu.VMEM((CH, D), jnp.float32),
            pltpu.VMEM((pt, CH), jnp.int32),
            pltpu.VMEM_SHARED((N, D), jnp.float32),
            pltpu.SemaphoreType.DMA,
            pltpu.SemaphoreType.DMA,
            pltpu.SemaphoreType.DMA,
            pltpu.SemaphoreType.DMA,
        ],
    )
    idx2d = jnp.pad(idx.reshape(nblocks, CH), ((0, NS * pt - nblocks), (0, 0)))
    return call(msg_s, msg_e, base_s, base_e, idx2d)


def kernel(node_invariant, node_equivariant, rbf, fcut, rsh, edge_index,
           W_q, W_k, W_v, W_qs, W_ks, W_vs,
           rbf_w1, rbf_b1, rbf_w2, rbf_b2,
           inv_w1, inv_b1, inv_w2, inv_b2):
    center = edge_index[0]
    neighbor = edge_index[1]
    ctr_tab, nbr_tab = _proj(node_invariant, W_q, W_qs, W_k, W_v, W_ks, W_vs)
    ctr_rows = _gather(ctr_tab, center, CTR_W, 128)
    nbr_rows = _gather(nbr_tab, neighbor, NBR_W, 64)
    weights = (inv_w1, inv_b1.reshape(1, D), inv_w2, inv_b2.reshape(1, D),
               rbf_w1, rbf_b1.reshape(1, D), rbf_w2, rbf_b2.reshape(1, D),
               jnp.asarray(_S), jnp.asarray(_B32), jnp.asarray(_P),
               jnp.asarray(_R))
    msg_s, msg_e = _edge(ctr_rows, nbr_rows, rbf, fcut, rsh, weights)
    out_s, out_e = _scatter(msg_s, msg_e, node_invariant, node_equivariant, center)
    return out_s, out_e


# speedup vs baseline: 5.3710x; 1.2561x over previous
import functools
import math

import jax
import jax.numpy as jnp
import numpy as np
from jax import lax
from jax.experimental import pallas as pl
from jax.experimental.pallas import tpu as pltpu
from jax.experimental.pallas import tpu_sc as plsc

N = 10000
E = 160000
D = 128
MUL = 32
NB = 20
CTR_W = 256
NBR_W = 384

NC = 2
NS = 16
CH = 128
NCHUNK = E // CH


def _const_mats():
    S = np.zeros((D, 64), np.float32)
    for m in range(MUL):
        S[m, m] = 1.0
        for t in range(3):
            S[MUL + 3 * m + t, MUL + m] = 1.0
    B32 = np.zeros((D, D), np.float32)
    for i in range(D):
        for j in range(D):
            if i // 32 == j // 32:
                B32[i, j] = 1.0
    P = np.zeros((D, 64), np.float32)
    for i in range(D):
        for j in range(64):
            if i // 64 == j // 32:
                P[i, j] = 1.0
    R = np.zeros((64, D), np.float32)
    for k in range(D):
        R[k if k < MUL else MUL + (k - MUL) // 3, k] = 1.0
    return S, B32, P, R


_S, _B32, _P, _R = _const_mats()


def _pack16(a, b):
    ua = lax.bitcast_convert_type(a, jnp.uint32)
    ub = lax.bitcast_convert_type(b, jnp.uint32)
    hi = (ua + jnp.uint32(0x8000)) & jnp.uint32(0xFFFF0000)
    lo = (ub + jnp.uint32(0x8000)) >> jnp.uint32(16)
    return lax.bitcast_convert_type(hi | lo, jnp.int32)


def _unpack_hi(p):
    u = lax.bitcast_convert_type(p, jnp.uint32)
    return lax.bitcast_convert_type(u & jnp.uint32(0xFFFF0000), jnp.float32)


def _unpack_lo(p):
    u = lax.bitcast_convert_type(p, jnp.uint32)
    return lax.bitcast_convert_type(u << jnp.uint32(16), jnp.float32)


def _proj_body(x_ref, wq_ref, wqs_ref, wk_ref, wv_ref, wks_ref, wvs_ref,
               ctr_ref, nbr_ref):
    x = x_ref[...]
    dn = (((1,), (1,)), ((), ()))
    dot = functools.partial(lax.dot_general, dimension_numbers=dn,
                            preferred_element_type=jnp.float32)
    ctr_ref[:, 0:D] = _pack16(x, dot(x, wq_ref[...]))
    ctr_ref[:, D:2 * D] = lax.bitcast_convert_type(dot(x, wqs_ref[...]),
                                                   jnp.int32)
    nbr_ref[:, 0:D] = _pack16(x, dot(x, wk_ref[...]))
    nbr_ref[:, D:2 * D] = _pack16(dot(x, wv_ref[...]), dot(x, wks_ref[...]))
    vs = lax.bitcast_convert_type(dot(x, wvs_ref[...]), jnp.int32)
    nbr_ref[:, 2 * D:2 * D + 64] = vs
    nbr_ref[:, 2 * D + 64:3 * D] = jnp.zeros_like(vs)


def _proj(x, wq, wqs, wk, wv, wks, wvs, *, interpret=False):
    bn = 1000
    grid = (N // bn,)
    full = lambda a: pl.BlockSpec(a.shape, lambda i: (0,) * a.ndim)
    return pl.pallas_call(
        _proj_body,
        grid=grid,
        in_specs=[pl.BlockSpec((bn, D), lambda i: (i, 0))] +
                 [full(w) for w in (wq, wqs, wk, wv, wks, wvs)],
        out_specs=[pl.BlockSpec((bn, CTR_W), lambda i: (i, 0)),
                   pl.BlockSpec((bn, NBR_W), lambda i: (i, 0))],
        out_shape=[jax.ShapeDtypeStruct((N, CTR_W), jnp.int32),
                   jax.ShapeDtypeStruct((N, NBR_W), jnp.int32)],
        interpret=interpret,
    )(x, wq, wqs, wk, wv, wks, wvs)


def _gather_body(nblocks, pw, ch, table_ref, idx2d_hbm, out_ref,
                 idxb, rows0, rows1, g0, g1, w0, w1):
    nw = NC * NS
    wid = lax.axis_index("c") * NS + lax.axis_index("s")
    c0 = wid * pw
    last_cnt = nblocks - (nw - 1) * pw
    cnt = jnp.where(wid < nw - 1, pw, last_cnt)

    pltpu.sync_copy(idx2d_hbm.at[pl.ds(c0, pw)], idxb)

    rows = (rows0, rows1)
    gs = (g0, g1)
    ws = (w0, w1)

    def step(i2, carry):
        for b in range(2):
            i = 2 * i2 + b

            @pl.when(i < cnt)
            def _(b=b, i=i):
                @pl.when(i2 > 0)
                def _():
                    pltpu.make_async_copy(
                        rows[b], out_ref.at[pl.ds(0, ch)], ws[b]).wait()

                pltpu.async_copy(table_ref.at[idxb.at[i]], rows[b], gs[b])

        for b in range(2):
            i = 2 * i2 + b

            @pl.when(i < cnt)
            def _(b=b, i=i):
                pltpu.make_async_copy(
                    table_ref.at[idxb.at[i]], rows[b], gs[b]).wait()
                pltpu.async_copy(rows[b], out_ref.at[pl.ds((c0 + i) * ch, ch)],
                                 ws[b])

        return carry

    lax.fori_loop(0, pw // 2, step, 0)

    for b in range(2):
        @pl.when(b < cnt)
        def _(b=b):
            pltpu.make_async_copy(rows[b], out_ref.at[pl.ds(0, ch)],
                                  ws[b]).wait()


def _gather(table, idx, width, ch):
    mesh = plsc.VectorSubcoreMesh(core_axis_name="c", subcore_axis_name="s")
    nblocks = E // ch
    pw = -(-nblocks // (NC * NS))
    pw += pw % 2
    call = pl.kernel(
        functools.partial(_gather_body, nblocks, pw, ch),
        out_type=jax.ShapeDtypeStruct((E, width), jnp.int32),
        mesh=mesh,
        scratch_types=[
            pltpu.VMEM((pw, ch), jnp.int32),
            pltpu.VMEM((ch, width), jnp.int32),
            pltpu.VMEM((ch, width), jnp.int32),
            pltpu.SemaphoreType.DMA,
            pltpu.SemaphoreType.DMA,
            pltpu.SemaphoreType.DMA,
            pltpu.SemaphoreType.DMA,
        ],
    )
    idx2d = jnp.pad(idx.reshape(nblocks, ch),
                    ((0, NC * NS * pw - nblocks), (0, 0)))
    return call(table, idx2d)


def _edge_body(ctr_ref, nbr_ref, rbf_ref, fcut_ref, rsh_ref,
               iw1_ref, ib1_ref, iw2_ref, ib2_ref,
               rw1_ref, rb1_ref, rw2_ref, rb2_ref,
               s_ref, b32_ref, p_ref, r_ref,
               msgs_ref, msge_ref):
    dn = (((1,), (1,)), ((), ()))
    dotT = functools.partial(lax.dot_general, dimension_numbers=dn,
                             preferred_element_type=jnp.float32)
    dot = functools.partial(lax.dot_general,
                            dimension_numbers=(((1,), (0,)), ((), ())),
                            preferred_element_type=jnp.float32)
    ctr = ctr_ref[...]
    nbr = nbr_ref[...]
    fcut = fcut_ref[...]
    x_i = _unpack_hi(ctr[:, 0:D])
    q_inv_c = _unpack_lo(ctr[:, 0:D])
    q_sph_c = lax.bitcast_convert_type(ctr[:, D:2 * D], jnp.float32)
    x_j = _unpack_hi(nbr[:, 0:D])
    k_inv_n = _unpack_lo(nbr[:, 0:D])
    v_inv_n = _unpack_hi(nbr[:, D:2 * D])
    k_sph_n = _unpack_lo(nbr[:, D:2 * D])
    v_sph_n = lax.bitcast_convert_type(nbr[:, 2 * D:2 * D + 64], jnp.float32)
    x_ij = x_j - x_i
    inv_x = dot(x_ij * x_ij, s_ref[...])
    h = dotT(inv_x, iw1_ref[...]) + ib1_ref[...]
    h = h * jax.nn.sigmoid(h)
    w_l = dotT(h, iw2_ref[...]) + ib2_ref[...]
    g = dotT(rbf_ref[...], rw1_ref[...]) + rb1_ref[...]
    g = g * jax.nn.sigmoid(g)
    w_r = dotT(g, rw2_ref[...]) + rb2_ref[...]
    w_ij = (w_l + w_r) * fcut
    t = (q_inv_c * w_ij) * k_inv_n
    attn_bc = dot(t, b32_ref[...]) * (1.0 / math.sqrt(D))
    msgs_ref[...] = attn_bc * v_inv_n
    t2 = (q_sph_c * w_ij) * k_sph_n
    gate = dot(t2, p_ref[...]) * (1.0 / math.sqrt(64)) * v_sph_n
    msge_ref[...] = dot(gate, r_ref[...]) * rsh_ref[...] * fcut


def _edge(ctr_rows, nbr_rows, rbf, fcut, rsh, weights, *, interpret=False):
    be = 1000
    grid = (E // be,)
    full = lambda a: pl.BlockSpec(a.shape, lambda i: (0,) * a.ndim)
    row = lambda w: pl.BlockSpec((be, w), lambda i: (i, 0))
    return pl.pallas_call(
        _edge_body,
        grid=grid,
        in_specs=[row(CTR_W), row(NBR_W), row(NB), row(1), row(D)] +
                 [full(w) for w in weights],
        out_specs=[row(D), row(D)],
        out_shape=[jax.ShapeDtypeStruct((E, D), jnp.float32),
                   jax.ShapeDtypeStruct((E, D), jnp.float32)],
        interpret=interpret,
    )(ctr_rows, nbr_rows, rbf, fcut, rsh, *weights)


def _scatter_body(nblocks, pt, msgs_hbm, msge_hbm, bases_hbm, basee_hbm,
                  idx2d_hbm, outs_ref, oute_ref, msg0, msg1, idxb, acc,
                  m0, m1, s0, s1):
    cid = lax.axis_index("c")
    sid = lax.axis_index("s")
    rows = 632
    last_r0 = (NS - 1) * rows
    last_rows = N - last_r0
    c0 = sid * pt
    last_cnt = nblocks - (NS - 1) * pt
    cnt = jnp.where(sid < NS - 1, pt, last_cnt)
    msg = (msg0, msg1)
    ms = (m0, m1)
    ss = (s0, s1)

    def run(msg_hbm, base_hbm, out_ref):
        def slab_copy(src, dst):
            @pl.when(sid < NS - 1)
            def _():
                s = pl.ds(sid * rows, rows)
                pltpu.sync_copy(src.at[s], dst.at[s])

            @pl.when(sid == NS - 1)
            def _():
                s = pl.ds(last_r0, last_rows)
                pltpu.sync_copy(src.at[s], dst.at[s])

        pltpu.sync_copy(idx2d_hbm.at[pl.ds(c0, pt)], idxb)

        slab_copy(base_hbm, acc)
        plsc.subcore_barrier()

        def step(i2, carry):
            for b in range(2):
                i = 2 * i2 + b

                @pl.when(i < cnt)
                def _(b=b, i=i):
                    @pl.when(i2 > 0)
                    def _():
                        pltpu.make_async_copy(
                            msg[b], acc.at[idxb.at[0]], ss[b]).wait()

                    pltpu.async_copy(msg_hbm.at[pl.ds((c0 + i) * CH, CH)],
                                     msg[b], ms[b])

            for b in range(2):
                i = 2 * i2 + b

                @pl.when(i < cnt)
                def _(b=b, i=i):
                    pltpu.make_async_copy(
                        msg_hbm.at[pl.ds(0, CH)], msg[b], ms[b]).wait()
                    pltpu.async_copy(msg[b], acc.at[idxb.at[i]], ss[b],
                                     add=True)

            return carry

        lax.fori_loop(0, pt // 2, step, 0)

        for b in range(2):
            @pl.when(b < cnt)
            def _(b=b):
                pltpu.make_async_copy(msg[b], acc.at[idxb.at[0]], ss[b]).wait()

        plsc.subcore_barrier()
        slab_copy(acc, out_ref)

    @pl.when(cid == 0)
    def _():
        run(msgs_hbm, bases_hbm, outs_ref)

    @pl.when(cid == 1)
    def _():
        run(msge_hbm, basee_hbm, oute_ref)


def _scatter(msg_s, msg_e, base_s, base_e, idx):
    mesh = plsc.VectorSubcoreMesh(core_axis_name="c", subcore_axis_name="s")
    nblocks = E // CH
    pt = -(-nblocks // NS)
    pt += pt % 2
    call = pl.kernel(
        functools.partial(_scatter_body, nblocks, pt),
        out_type=[jax.ShapeDtypeStruct((N, D), jnp.float32),
                  jax.ShapeDtypeStruct((N, D), jnp.float32)],
        mesh=mesh,
        scratch_types=[
            pltpu.VMEM((CH, D), jnp.float32),
            pltpu.VMEM((CH, D), jnp.float32),
            pltpu.VMEM((pt, CH), jnp.int32),
            pltpu.VMEM_SHARED((N, D), jnp.float32),
            pltpu.SemaphoreType.DMA,
            pltpu.SemaphoreType.DMA,
            pltpu.SemaphoreType.DMA,
            pltpu.SemaphoreType.DMA,
        ],
    )
    idx2d = jnp.pad(idx.reshape(nblocks, CH), ((0, NS * pt - nblocks), (0, 0)))
    return call(msg_s, msg_e, base_s, base_e, idx2d)


def kernel(node_invariant, node_equivariant, rbf, fcut, rsh, edge_index,
           W_q, W_k, W_v, W_qs, W_ks, W_vs,
           rbf_w1, rbf_b1, rbf_w2, rbf_b2,
           inv_w1, inv_b1, inv_w2, inv_b2):
    center = edge_index[0]
    neighbor = edge_index[1]
    ctr_tab, nbr_tab = _proj(node_invariant, W_q, W_qs, W_k, W_v, W_ks, W_vs)
    ctr_rows = _gather(ctr_tab, center, CTR_W, 128)
    nbr_rows = _gather(nbr_tab, neighbor, NBR_W, 128)
    weights = (inv_w1, inv_b1.reshape(1, D), inv_w2, inv_b2.reshape(1, D),
               rbf_w1, rbf_b1.reshape(1, D), rbf_w2, rbf_b2.reshape(1, D),
               jnp.asarray(_S), jnp.asarray(_B32), jnp.asarray(_P),
               jnp.asarray(_R))
    msg_s, msg_e = _edge(ctr_rows, nbr_rows, rbf, fcut, rsh, weights)
    out_s, out_e = _scatter(msg_s, msg_e, node_invariant, node_equivariant, center)
    return out_s, out_e


# speedup vs baseline: 5.8021x; 1.0803x over previous
import functools
import math

import jax
import jax.numpy as jnp
import numpy as np
from jax import lax
from jax.experimental import pallas as pl
from jax.experimental.pallas import tpu as pltpu
from jax.experimental.pallas import tpu_sc as plsc

N = 10000
E = 160000
D = 128
MUL = 32
NB = 20
CTR_W = 256
NBR_W = 384

NC = 2
NS = 16
CH = 128
NCHUNK = E // CH


def _const_mats():
    S = np.zeros((D, 64), np.float32)
    for m in range(MUL):
        S[m, m] = 1.0
        for t in range(3):
            S[MUL + 3 * m + t, MUL + m] = 1.0
    B32 = np.zeros((D, D), np.float32)
    for i in range(D):
        for j in range(D):
            if i // 32 == j // 32:
                B32[i, j] = 1.0
    P = np.zeros((D, 64), np.float32)
    for i in range(D):
        for j in range(64):
            if i // 64 == j // 32:
                P[i, j] = 1.0
    R = np.zeros((64, D), np.float32)
    for k in range(D):
        R[k if k < MUL else MUL + (k - MUL) // 3, k] = 1.0
    return S, B32, P, R


_S, _B32, _P, _R = _const_mats()


def _pack16(a, b):
    ua = lax.bitcast_convert_type(a, jnp.uint32)
    ub = lax.bitcast_convert_type(b, jnp.uint32)
    hi = (ua + jnp.uint32(0x8000)) & jnp.uint32(0xFFFF0000)
    lo = (ub + jnp.uint32(0x8000)) >> jnp.uint32(16)
    return lax.bitcast_convert_type(hi | lo, jnp.int32)


def _unpack_hi(p):
    u = lax.bitcast_convert_type(p, jnp.uint32)
    return lax.bitcast_convert_type(u & jnp.uint32(0xFFFF0000), jnp.float32)


def _unpack_lo(p):
    u = lax.bitcast_convert_type(p, jnp.uint32)
    return lax.bitcast_convert_type(u << jnp.uint32(16), jnp.float32)


def _proj_body(x_ref, wq_ref, wqs_ref, wk_ref, wv_ref, wks_ref, wvs_ref,
               ctr_ref, nbr_ref):
    x = x_ref[...]
    dn = (((1,), (1,)), ((), ()))
    dot = functools.partial(lax.dot_general, dimension_numbers=dn,
                            preferred_element_type=jnp.float32)
    ctr_ref[:, 0:D] = _pack16(x, dot(x, wq_ref[...]))
    ctr_ref[:, D:2 * D] = lax.bitcast_convert_type(dot(x, wqs_ref[...]),
                                                   jnp.int32)
    nbr_ref[:, 0:D] = _pack16(x, dot(x, wk_ref[...]))
    nbr_ref[:, D:2 * D] = _pack16(dot(x, wv_ref[...]), dot(x, wks_ref[...]))
    vs = lax.bitcast_convert_type(dot(x, wvs_ref[...]), jnp.int32)
    nbr_ref[:, 2 * D:2 * D + 64] = vs
    nbr_ref[:, 2 * D + 64:3 * D] = jnp.zeros_like(vs)


def _proj(x, wq, wqs, wk, wv, wks, wvs, *, interpret=False):
    bn = 1000
    grid = (N // bn,)
    full = lambda a: pl.BlockSpec(a.shape, lambda i: (0,) * a.ndim)
    return pl.pallas_call(
        _proj_body,
        grid=grid,
        in_specs=[pl.BlockSpec((bn, D), lambda i: (i, 0))] +
                 [full(w) for w in (wq, wqs, wk, wv, wks, wvs)],
        out_specs=[pl.BlockSpec((bn, CTR_W), lambda i: (i, 0)),
                   pl.BlockSpec((bn, NBR_W), lambda i: (i, 0))],
        out_shape=[jax.ShapeDtypeStruct((N, CTR_W), jnp.int32),
                   jax.ShapeDtypeStruct((N, NBR_W), jnp.int32)],
        interpret=interpret,
    )(x, wq, wqs, wk, wv, wks, wvs)


def _gather_body(nblocks, pw, ch, table_ref, idx2d_hbm, out_ref,
                 idxb, rows0, rows1, g0, g1, w0, w1):
    wid = lax.axis_index("c") * NS + lax.axis_index("s")
    c0 = wid * pw
    cnt = jnp.clip(nblocks - c0, 0, pw)

    pltpu.sync_copy(idx2d_hbm.at[pl.ds(c0, pw)], idxb)

    rows = (rows0, rows1)
    gs = (g0, g1)
    ws = (w0, w1)

    def step(i2, carry):
        for b in range(2):
            i = 2 * i2 + b

            @pl.when(i < cnt)
            def _(b=b, i=i):
                @pl.when(i2 > 0)
                def _():
                    pltpu.make_async_copy(
                        rows[b], out_ref.at[pl.ds(0, ch)], ws[b]).wait()

                pltpu.async_copy(table_ref.at[idxb.at[i]], rows[b], gs[b])

        for b in range(2):
            i = 2 * i2 + b

            @pl.when(i < cnt)
            def _(b=b, i=i):
                pltpu.make_async_copy(
                    table_ref.at[idxb.at[i]], rows[b], gs[b]).wait()
                pltpu.async_copy(rows[b], out_ref.at[pl.ds((c0 + i) * ch, ch)],
                                 ws[b])

        return carry

    lax.fori_loop(0, pw // 2, step, 0)

    for b in range(2):
        @pl.when(b < cnt)
        def _(b=b):
            pltpu.make_async_copy(rows[b], out_ref.at[pl.ds(0, ch)],
                                  ws[b]).wait()


def _gather(table, idx, width, ch):
    mesh = plsc.VectorSubcoreMesh(core_axis_name="c", subcore_axis_name="s")
    nblocks = idx.shape[0] // ch
    pw = -(-nblocks // (NC * NS))
    pw = -(-pw // 8) * 8
    call = pl.kernel(
        functools.partial(_gather_body, nblocks, pw, ch),
        out_type=jax.ShapeDtypeStruct((idx.shape[0], width), jnp.int32),
        mesh=mesh,
        scratch_types=[
            pltpu.VMEM((pw, ch), jnp.int32),
            pltpu.VMEM((ch, width), jnp.int32),
            pltpu.VMEM((ch, width), jnp.int32),
            pltpu.SemaphoreType.DMA,
            pltpu.SemaphoreType.DMA,
            pltpu.SemaphoreType.DMA,
            pltpu.SemaphoreType.DMA,
        ],
    )
    idx2d = jnp.pad(idx.reshape(nblocks, ch),
                    ((0, NC * NS * pw - nblocks), (0, 0)))
    return call(table, idx2d)


def _edge_body(ctr_ref, nbr_ref, rbf_ref, fcut_ref, rsh_ref,
               iw1_ref, ib1_ref, iw2_ref, ib2_ref,
               rw1_ref, rb1_ref, rw2_ref, rb2_ref,
               s_ref, b32_ref, p_ref, r_ref,
               msgs_ref, msge_ref):
    dn = (((1,), (1,)), ((), ()))
    dotT = functools.partial(lax.dot_general, dimension_numbers=dn,
                             preferred_element_type=jnp.float32)
    dot = functools.partial(lax.dot_general,
                            dimension_numbers=(((1,), (0,)), ((), ())),
                            preferred_element_type=jnp.float32)
    ctr = ctr_ref[...]
    nbr = nbr_ref[...]
    fcut = fcut_ref[...]
    x_i = _unpack_hi(ctr[:, 0:D])
    q_inv_c = _unpack_lo(ctr[:, 0:D])
    q_sph_c = lax.bitcast_convert_type(ctr[:, D:2 * D], jnp.float32)
    x_j = _unpack_hi(nbr[:, 0:D])
    k_inv_n = _unpack_lo(nbr[:, 0:D])
    v_inv_n = _unpack_hi(nbr[:, D:2 * D])
    k_sph_n = _unpack_lo(nbr[:, D:2 * D])
    v_sph_n = lax.bitcast_convert_type(nbr[:, 2 * D:2 * D + 64], jnp.float32)
    x_ij = x_j - x_i
    inv_x = dot(x_ij * x_ij, s_ref[...])
    h = dotT(inv_x, iw1_ref[...]) + ib1_ref[...]
    h = h * jax.nn.sigmoid(h)
    w_l = dotT(h, iw2_ref[...]) + ib2_ref[...]
    g = dotT(rbf_ref[...], rw1_ref[...]) + rb1_ref[...]
    g = g * jax.nn.sigmoid(g)
    w_r = dotT(g, rw2_ref[...]) + rb2_ref[...]
    w_ij = (w_l + w_r) * fcut
    t = (q_inv_c * w_ij) * k_inv_n
    attn_bc = dot(t, b32_ref[...]) * (1.0 / math.sqrt(D))
    msgs_ref[...] = attn_bc * v_inv_n
    t2 = (q_sph_c * w_ij) * k_sph_n
    gate = dot(t2, p_ref[...]) * (1.0 / math.sqrt(64)) * v_sph_n
    msge_ref[...] = dot(gate, r_ref[...]) * rsh_ref[...] * fcut


def _edge(ctr_rows, nbr_rows, rbf, fcut, rsh, weights, off=0, *,
          interpret=False):
    be = 1000
    ne = ctr_rows.shape[0]
    grid = (ne // be,)
    ob = off // be
    full = lambda a: pl.BlockSpec(a.shape, lambda i: (0,) * a.ndim)
    row = lambda w: pl.BlockSpec((be, w), lambda i: (i, 0))
    rowo = lambda w: pl.BlockSpec((be, w), lambda i: (i + ob, 0))
    return pl.pallas_call(
        _edge_body,
        grid=grid,
        in_specs=[row(CTR_W), row(NBR_W), rowo(NB), rowo(1), rowo(D)] +
                 [full(w) for w in weights],
        out_specs=[row(D), row(D)],
        out_shape=[jax.ShapeDtypeStruct((ne, D), jnp.float32),
                   jax.ShapeDtypeStruct((ne, D), jnp.float32)],
        interpret=interpret,
    )(ctr_rows, nbr_rows, rbf, fcut, rsh, *weights)


def _scatter_body(nblocks, pt, msgs_hbm, msge_hbm, bases_hbm, basee_hbm,
                  idx2d_hbm, outs_ref, oute_ref, msg0, msg1, idxb, acc,
                  m0, m1, s0, s1):
    cid = lax.axis_index("c")
    sid = lax.axis_index("s")
    rows = 632
    last_r0 = (NS - 1) * rows
    last_rows = N - last_r0
    c0 = sid * pt
    cnt = jnp.clip(nblocks - c0, 0, pt)
    msg = (msg0, msg1)
    ms = (m0, m1)
    ss = (s0, s1)

    def run(msg_hbm, base_hbm, out_ref):
        def slab_copy(src, dst):
            @pl.when(sid < NS - 1)
            def _():
                s = pl.ds(sid * rows, rows)
                pltpu.sync_copy(src.at[s], dst.at[s])

            @pl.when(sid == NS - 1)
            def _():
                s = pl.ds(last_r0, last_rows)
                pltpu.sync_copy(src.at[s], dst.at[s])

        pltpu.sync_copy(idx2d_hbm.at[pl.ds(c0, pt)], idxb)

        slab_copy(base_hbm, acc)
        plsc.subcore_barrier()

        def step(i2, carry):
            for b in range(2):
                i = 2 * i2 + b

                @pl.when(i < cnt)
                def _(b=b, i=i):
                    @pl.when(i2 > 0)
                    def _():
                        pltpu.make_async_copy(
                            msg[b], acc.at[idxb.at[0]], ss[b]).wait()

                    pltpu.async_copy(msg_hbm.at[pl.ds((c0 + i) * CH, CH)],
                                     msg[b], ms[b])

            for b in range(2):
                i = 2 * i2 + b

                @pl.when(i < cnt)
                def _(b=b, i=i):
                    pltpu.make_async_copy(
                        msg_hbm.at[pl.ds(0, CH)], msg[b], ms[b]).wait()
                    pltpu.async_copy(msg[b], acc.at[idxb.at[i]], ss[b],
                                     add=True)

            return carry

        lax.fori_loop(0, pt // 2, step, 0)

        for b in range(2):
            @pl.when(b < cnt)
            def _(b=b):
                pltpu.make_async_copy(msg[b], acc.at[idxb.at[0]], ss[b]).wait()

        plsc.subcore_barrier()
        slab_copy(acc, out_ref)

    @pl.when(cid == 0)
    def _():
        run(msgs_hbm, bases_hbm, outs_ref)

    @pl.when(cid == 1)
    def _():
        run(msge_hbm, basee_hbm, oute_ref)


def _scatter(msg_s, msg_e, base_s, base_e, idx):
    mesh = plsc.VectorSubcoreMesh(core_axis_name="c", subcore_axis_name="s")
    nblocks = idx.shape[0] // CH
    pt = -(-nblocks // NS)
    pt = -(-pt // 8) * 8
    call = pl.kernel(
        functools.partial(_scatter_body, nblocks, pt),
        out_type=[jax.ShapeDtypeStruct((N, D), jnp.float32),
                  jax.ShapeDtypeStruct((N, D), jnp.float32)],
        mesh=mesh,
        scratch_types=[
            pltpu.VMEM((CH, D), jnp.float32),
            pltpu.VMEM((CH, D), jnp.float32),
            pltpu.VMEM((pt, CH), jnp.int32),
            pltpu.VMEM_SHARED((N, D), jnp.float32),
            pltpu.SemaphoreType.DMA,
            pltpu.SemaphoreType.DMA,
            pltpu.SemaphoreType.DMA,
            pltpu.SemaphoreType.DMA,
        ],
    )
    idx2d = jnp.pad(idx.reshape(nblocks, CH), ((0, NS * pt - nblocks), (0, 0)))
    return call(msg_s, msg_e, base_s, base_e, idx2d)


def kernel(node_invariant, node_equivariant, rbf, fcut, rsh, edge_index,
           W_q, W_k, W_v, W_qs, W_ks, W_vs,
           rbf_w1, rbf_b1, rbf_w2, rbf_b2,
           inv_w1, inv_b1, inv_w2, inv_b2):
    center = edge_index[0]
    neighbor = edge_index[1]
    h = E // 2
    ctr_tab, nbr_tab = _proj(node_invariant, W_q, W_qs, W_k, W_v, W_ks, W_vs)
    weights = (inv_w1, inv_b1.reshape(1, D), inv_w2, inv_b2.reshape(1, D),
               rbf_w1, rbf_b1.reshape(1, D), rbf_w2, rbf_b2.reshape(1, D),
               jnp.asarray(_S), jnp.asarray(_B32), jnp.asarray(_P),
               jnp.asarray(_R))
    g1c = _gather(ctr_tab, center[:h], CTR_W, 128)
    g1n = _gather(nbr_tab, neighbor[:h], NBR_W, 128)
    g2c = _gather(ctr_tab, center[h:], CTR_W, 128)
    g2n = _gather(nbr_tab, neighbor[h:], NBR_W, 128)
    m1s, m1e = _edge(g1c, g1n, rbf, fcut, rsh, weights, 0)
    m2s, m2e = _edge(g2c, g2n, rbf, fcut, rsh, weights, h)
    s1, e1 = _scatter(m1s, m1e, node_invariant, node_equivariant, center[:h])
    s2, e2 = _scatter(m2s, m2e, s1, e1, center[h:])
    return s2, e2


# speedup vs baseline: 6.0725x; 1.0466x over previous
import functools
import math

import jax
import jax.numpy as jnp
import numpy as np
from jax import lax
from jax.experimental import pallas as pl
from jax.experimental.pallas import tpu as pltpu
from jax.experimental.pallas import tpu_sc as plsc

N = 10000
E = 160000
D = 128
MUL = 32
NB = 20
CTR_W = 256
NBR_W = 384

NC = 2
NS = 16
CH = 128
NCHUNK = E // CH


def _const_mats():
    S = np.zeros((D, 64), np.float32)
    for m in range(MUL):
        S[m, m] = 1.0
        for t in range(3):
            S[MUL + 3 * m + t, MUL + m] = 1.0
    B32 = np.zeros((D, D), np.float32)
    for i in range(D):
        for j in range(D):
            if i // 32 == j // 32:
                B32[i, j] = 1.0
    P = np.zeros((D, 64), np.float32)
    for i in range(D):
        for j in range(64):
            if i // 64 == j // 32:
                P[i, j] = 1.0
    R = np.zeros((64, D), np.float32)
    for k in range(D):
        R[k if k < MUL else MUL + (k - MUL) // 3, k] = 1.0
    return S, B32, P, R


_S, _B32, _P, _R = _const_mats()


def _pack16(a, b):
    ua = lax.bitcast_convert_type(a, jnp.uint32)
    ub = lax.bitcast_convert_type(b, jnp.uint32)
    hi = (ua + jnp.uint32(0x8000)) & jnp.uint32(0xFFFF0000)
    lo = (ub + jnp.uint32(0x8000)) >> jnp.uint32(16)
    return lax.bitcast_convert_type(hi | lo, jnp.int32)


def _unpack_hi(p):
    u = lax.bitcast_convert_type(p, jnp.uint32)
    return lax.bitcast_convert_type(u & jnp.uint32(0xFFFF0000), jnp.float32)


def _unpack_lo(p):
    u = lax.bitcast_convert_type(p, jnp.uint32)
    return lax.bitcast_convert_type(u << jnp.uint32(16), jnp.float32)


def _proj_body(x_ref, wq_ref, wqs_ref, wk_ref, wv_ref, wks_ref, wvs_ref,
               ctr_ref, nbr_ref):
    x = x_ref[...]
    dn = (((1,), (1,)), ((), ()))
    dot = functools.partial(lax.dot_general, dimension_numbers=dn,
                            preferred_element_type=jnp.float32)
    ctr_ref[:, 0:D] = _pack16(x, dot(x, wq_ref[...]))
    ctr_ref[:, D:2 * D] = lax.bitcast_convert_type(dot(x, wqs_ref[...]),
                                                   jnp.int32)
    nbr_ref[:, 0:D] = _pack16(x, dot(x, wk_ref[...]))
    nbr_ref[:, D:2 * D] = _pack16(dot(x, wv_ref[...]), dot(x, wks_ref[...]))
    vs = lax.bitcast_convert_type(dot(x, wvs_ref[...]), jnp.int32)
    nbr_ref[:, 2 * D:2 * D + 64] = vs
    nbr_ref[:, 2 * D + 64:3 * D] = jnp.zeros_like(vs)


def _proj(x, wq, wqs, wk, wv, wks, wvs, *, interpret=False):
    bn = 1000
    grid = (N // bn,)
    full = lambda a: pl.BlockSpec(a.shape, lambda i: (0,) * a.ndim)
    return pl.pallas_call(
        _proj_body,
        grid=grid,
        in_specs=[pl.BlockSpec((bn, D), lambda i: (i, 0))] +
                 [full(w) for w in (wq, wqs, wk, wv, wks, wvs)],
        out_specs=[pl.BlockSpec((bn, CTR_W), lambda i: (i, 0)),
                   pl.BlockSpec((bn, NBR_W), lambda i: (i, 0))],
        out_shape=[jax.ShapeDtypeStruct((N, CTR_W), jnp.int32),
                   jax.ShapeDtypeStruct((N, NBR_W), jnp.int32)],
        interpret=interpret,
    )(x, wq, wqs, wk, wv, wks, wvs)


def _gather_body(nblocks, pw, ch, table_ref, idx2d_hbm, out_ref,
                 idxb, rows0, rows1, g0, g1, w0, w1):
    wid = lax.axis_index("c") * NS + lax.axis_index("s")
    c0 = wid * pw
    cnt = jnp.clip(nblocks - c0, 0, pw)

    pltpu.sync_copy(idx2d_hbm.at[pl.ds(c0, pw)], idxb)

    rows = (rows0, rows1)
    gs = (g0, g1)
    ws = (w0, w1)

    def step(i2, carry):
        for b in range(2):
            i = 2 * i2 + b

            @pl.when(i < cnt)
            def _(b=b, i=i):
                @pl.when(i2 > 0)
                def _():
                    pltpu.make_async_copy(
                        rows[b], out_ref.at[pl.ds(0, ch)], ws[b]).wait()

                pltpu.async_copy(table_ref.at[idxb.at[i]], rows[b], gs[b])

        for b in range(2):
            i = 2 * i2 + b

            @pl.when(i < cnt)
            def _(b=b, i=i):
                pltpu.make_async_copy(
                    table_ref.at[idxb.at[i]], rows[b], gs[b]).wait()
                pltpu.async_copy(rows[b], out_ref.at[pl.ds((c0 + i) * ch, ch)],
                                 ws[b])

        return carry

    lax.fori_loop(0, pw // 2, step, 0)

    for b in range(2):
        @pl.when(b < cnt)
        def _(b=b):
            pltpu.make_async_copy(rows[b], out_ref.at[pl.ds(0, ch)],
                                  ws[b]).wait()


def _gather(table, idx, width, ch):
    mesh = plsc.VectorSubcoreMesh(core_axis_name="c", subcore_axis_name="s")
    nblocks = idx.shape[0] // ch
    pw = -(-nblocks // (NC * NS))
    pw = -(-pw // 8) * 8
    call = pl.kernel(
        functools.partial(_gather_body, nblocks, pw, ch),
        out_type=jax.ShapeDtypeStruct((idx.shape[0], width), jnp.int32),
        mesh=mesh,
        scratch_types=[
            pltpu.VMEM((pw, ch), jnp.int32),
            pltpu.VMEM((ch, width), jnp.int32),
            pltpu.VMEM((ch, width), jnp.int32),
            pltpu.SemaphoreType.DMA,
            pltpu.SemaphoreType.DMA,
            pltpu.SemaphoreType.DMA,
            pltpu.SemaphoreType.DMA,
        ],
    )
    idx2d = jnp.pad(idx.reshape(nblocks, ch),
                    ((0, NC * NS * pw - nblocks), (0, 0)))
    return call(table, idx2d)


def _gather2_body(nblocks, pw, ch, ctab_ref, ntab_ref, idxc_hbm, idxn_hbm,
                  outc_ref, outn_ref, idxcb, idxnb,
                  c_r0, c_r1, n_r0, n_r1,
                  gc0, gc1, gn0, gn1, wc0, wc1, wn0, wn1):
    wid = lax.axis_index("c") * NS + lax.axis_index("s")
    c0 = wid * pw
    cnt = jnp.clip(nblocks - c0, 0, pw)

    pltpu.sync_copy(idxc_hbm.at[pl.ds(c0, pw)], idxcb)
    pltpu.sync_copy(idxn_hbm.at[pl.ds(c0, pw)], idxnb)

    cr = (c_r0, c_r1)
    nr = (n_r0, n_r1)
    gc = (gc0, gc1)
    gn = (gn0, gn1)
    wc = (wc0, wc1)
    wn = (wn0, wn1)

    def step(i2, carry):
        for b in range(2):
            i = 2 * i2 + b

            @pl.when(i < cnt)
            def _(b=b, i=i):
                @pl.when(i2 > 0)
                def _():
                    pltpu.make_async_copy(
                        cr[b], outc_ref.at[pl.ds(0, ch)], wc[b]).wait()
                    pltpu.make_async_copy(
                        nr[b], outn_ref.at[pl.ds(0, ch)], wn[b]).wait()

                pltpu.async_copy(ctab_ref.at[idxcb.at[i]], cr[b], gc[b])
                pltpu.async_copy(ntab_ref.at[idxnb.at[i]], nr[b], gn[b])

        for b in range(2):
            i = 2 * i2 + b

            @pl.when(i < cnt)
            def _(b=b, i=i):
                pltpu.make_async_copy(
                    ctab_ref.at[idxcb.at[i]], cr[b], gc[b]).wait()
                pltpu.async_copy(cr[b], outc_ref.at[pl.ds((c0 + i) * ch, ch)],
                                 wc[b])
                pltpu.make_async_copy(
                    ntab_ref.at[idxnb.at[i]], nr[b], gn[b]).wait()
                pltpu.async_copy(nr[b], outn_ref.at[pl.ds((c0 + i) * ch, ch)],
                                 wn[b])

        return carry

    lax.fori_loop(0, pw // 2, step, 0)

    for b in range(2):
        @pl.when(b < cnt)
        def _(b=b):
            pltpu.make_async_copy(cr[b], outc_ref.at[pl.ds(0, ch)],
                                  wc[b]).wait()
            pltpu.make_async_copy(nr[b], outn_ref.at[pl.ds(0, ch)],
                                  wn[b]).wait()


def _gather2(ctab, ntab, idxc, idxn, ch=64):
    mesh = plsc.VectorSubcoreMesh(core_axis_name="c", subcore_axis_name="s")
    ne = idxc.shape[0]
    nblocks = ne // ch
    pw = -(-nblocks // (NC * NS))
    pw = -(-pw // 8) * 8
    call = pl.kernel(
        functools.partial(_gather2_body, nblocks, pw, ch),
        out_type=[jax.ShapeDtypeStruct((ne, CTR_W), jnp.int32),
                  jax.ShapeDtypeStruct((ne, NBR_W), jnp.int32)],
        mesh=mesh,
        scratch_types=[
            pltpu.VMEM((pw, ch), jnp.int32),
            pltpu.VMEM((pw, ch), jnp.int32),
            pltpu.VMEM((ch, CTR_W), jnp.int32),
            pltpu.VMEM((ch, CTR_W), jnp.int32),
            pltpu.VMEM((ch, NBR_W), jnp.int32),
            pltpu.VMEM((ch, NBR_W), jnp.int32),
        ] + [pltpu.SemaphoreType.DMA] * 8,
    )
    pad = ((0, NC * NS * pw - nblocks), (0, 0))
    return call(ctab, ntab, jnp.pad(idxc.reshape(nblocks, ch), pad),
                jnp.pad(idxn.reshape(nblocks, ch), pad))


def _edge_body(ctr_ref, nbr_ref, rbf_ref, fcut_ref, rsh_ref,
               iw1_ref, ib1_ref, iw2_ref, ib2_ref,
               rw1_ref, rb1_ref, rw2_ref, rb2_ref,
               s_ref, b32_ref, p_ref, r_ref,
               msgs_ref, msge_ref):
    dn = (((1,), (1,)), ((), ()))
    dotT = functools.partial(lax.dot_general, dimension_numbers=dn,
                             preferred_element_type=jnp.float32)
    dot = functools.partial(lax.dot_general,
                            dimension_numbers=(((1,), (0,)), ((), ())),
                            preferred_element_type=jnp.float32)
    ctr = ctr_ref[...]
    nbr = nbr_ref[...]
    fcut = fcut_ref[...]
    x_i = _unpack_hi(ctr[:, 0:D])
    q_inv_c = _unpack_lo(ctr[:, 0:D])
    q_sph_c = lax.bitcast_convert_type(ctr[:, D:2 * D], jnp.float32)
    x_j = _unpack_hi(nbr[:, 0:D])
    k_inv_n = _unpack_lo(nbr[:, 0:D])
    v_inv_n = _unpack_hi(nbr[:, D:2 * D])
    k_sph_n = _unpack_lo(nbr[:, D:2 * D])
    v_sph_n = lax.bitcast_convert_type(nbr[:, 2 * D:2 * D + 64], jnp.float32)
    x_ij = x_j - x_i
    inv_x = dot(x_ij * x_ij, s_ref[...])
    h = dotT(inv_x, iw1_ref[...]) + ib1_ref[...]
    h = h * jax.nn.sigmoid(h)
    w_l = dotT(h, iw2_ref[...]) + ib2_ref[...]
    g = dotT(rbf_ref[...], rw1_ref[...]) + rb1_ref[...]
    g = g * jax.nn.sigmoid(g)
    w_r = dotT(g, rw2_ref[...]) + rb2_ref[...]
    w_ij = (w_l + w_r) * fcut
    t = (q_inv_c * w_ij) * k_inv_n
    attn_bc = dot(t, b32_ref[...]) * (1.0 / math.sqrt(D))
    msgs_ref[...] = attn_bc * v_inv_n
    t2 = (q_sph_c * w_ij) * k_sph_n
    gate = dot(t2, p_ref[...]) * (1.0 / math.sqrt(64)) * v_sph_n
    msge_ref[...] = dot(gate, r_ref[...]) * rsh_ref[...] * fcut


def _edge(ctr_rows, nbr_rows, rbf, fcut, rsh, weights, off=0, *,
          interpret=False):
    be = 2000
    ne = ctr_rows.shape[0]
    grid = (ne // be,)
    ob = off // be
    full = lambda a: pl.BlockSpec(a.shape, lambda i: (0,) * a.ndim)
    row = lambda w: pl.BlockSpec((be, w), lambda i: (i, 0))
    rowo = lambda w: pl.BlockSpec((be, w), lambda i: (i + ob, 0))
    return pl.pallas_call(
        _edge_body,
        grid=grid,
        in_specs=[row(CTR_W), row(NBR_W), rowo(NB), rowo(1), rowo(D)] +
                 [full(w) for w in weights],
        out_specs=[row(D), row(D)],
        out_shape=[jax.ShapeDtypeStruct((ne, D), jnp.float32),
                   jax.ShapeDtypeStruct((ne, D), jnp.float32)],
        interpret=interpret,
    )(ctr_rows, nbr_rows, rbf, fcut, rsh, *weights)


def _scatter_body(nblocks, pt, msgs_hbm, msge_hbm, bases_hbm, basee_hbm,
                  idx2d_hbm, outs_ref, oute_ref, msg0, msg1, idxb, acc,
                  m0, m1, s0, s1):
    cid = lax.axis_index("c")
    sid = lax.axis_index("s")
    rows = 632
    last_r0 = (NS - 1) * rows
    last_rows = N - last_r0
    c0 = sid * pt
    cnt = jnp.clip(nblocks - c0, 0, pt)
    msg = (msg0, msg1)
    ms = (m0, m1)
    ss = (s0, s1)

    def run(msg_hbm, base_hbm, out_ref):
        def slab_copy(src, dst):
            @pl.when(sid < NS - 1)
            def _():
                s = pl.ds(sid * rows, rows)
                pltpu.sync_copy(src.at[s], dst.at[s])

            @pl.when(sid == NS - 1)
            def _():
                s = pl.ds(last_r0, last_rows)
                pltpu.sync_copy(src.at[s], dst.at[s])

        pltpu.sync_copy(idx2d_hbm.at[pl.ds(c0, pt)], idxb)

        slab_copy(base_hbm, acc)
        plsc.subcore_barrier()

        def step(i2, carry):
            for b in range(2):
                i = 2 * i2 + b

                @pl.when(i < cnt)
                def _(b=b, i=i):
                    @pl.when(i2 > 0)
                    def _():
                        pltpu.make_async_copy(
                            msg[b], acc.at[idxb.at[0]], ss[b]).wait()

                    pltpu.async_copy(msg_hbm.at[pl.ds((c0 + i) * CH, CH)],
                                     msg[b], ms[b])

            for b in range(2):
                i = 2 * i2 + b

                @pl.when(i < cnt)
                def _(b=b, i=i):
                    pltpu.make_async_copy(
                        msg_hbm.at[pl.ds(0, CH)], msg[b], ms[b]).wait()
                    pltpu.async_copy(msg[b], acc.at[idxb.at[i]], ss[b],
                                     add=True)

            return carry

        lax.fori_loop(0, pt // 2, step, 0)

        for b in range(2):
            @pl.when(b < cnt)
            def _(b=b):
                pltpu.make_async_copy(msg[b], acc.at[idxb.at[0]], ss[b]).wait()

        plsc.subcore_barrier()
        slab_copy(acc, out_ref)

    @pl.when(cid == 0)
    def _():
        run(msgs_hbm, bases_hbm, outs_ref)

    @pl.when(cid == 1)
    def _():
        run(msge_hbm, basee_hbm, oute_ref)


def _scatter(msg_s, msg_e, base_s, base_e, idx):
    mesh = plsc.VectorSubcoreMesh(core_axis_name="c", subcore_axis_name="s")
    nblocks = idx.shape[0] // CH
    pt = -(-nblocks // NS)
    pt = -(-pt // 8) * 8
    call = pl.kernel(
        functools.partial(_scatter_body, nblocks, pt),
        out_type=[jax.ShapeDtypeStruct((N, D), jnp.float32),
                  jax.ShapeDtypeStruct((N, D), jnp.float32)],
        mesh=mesh,
        scratch_types=[
            pltpu.VMEM((CH, D), jnp.float32),
            pltpu.VMEM((CH, D), jnp.float32),
            pltpu.VMEM((pt, CH), jnp.int32),
            pltpu.VMEM_SHARED((N, D), jnp.float32),
            pltpu.SemaphoreType.DMA,
            pltpu.SemaphoreType.DMA,
            pltpu.SemaphoreType.DMA,
            pltpu.SemaphoreType.DMA,
        ],
    )
    idx2d = jnp.pad(idx.reshape(nblocks, CH), ((0, NS * pt - nblocks), (0, 0)))
    return call(msg_s, msg_e, base_s, base_e, idx2d)


def kernel(node_invariant, node_equivariant, rbf, fcut, rsh, edge_index,
           W_q, W_k, W_v, W_qs, W_ks, W_vs,
           rbf_w1, rbf_b1, rbf_w2, rbf_b2,
           inv_w1, inv_b1, inv_w2, inv_b2):
    center = edge_index[0]
    neighbor = edge_index[1]
    h = E // 2
    ctr_tab, nbr_tab = _proj(node_invariant, W_q, W_qs, W_k, W_v, W_ks, W_vs)
    weights = (inv_w1, inv_b1.reshape(1, D), inv_w2, inv_b2.reshape(1, D),
               rbf_w1, rbf_b1.reshape(1, D), rbf_w2, rbf_b2.reshape(1, D),
               jnp.asarray(_S), jnp.asarray(_B32), jnp.asarray(_P),
               jnp.asarray(_R))
    g1c, g1n = _gather2(ctr_tab, nbr_tab, center[:h], neighbor[:h])
    g2c, g2n = _gather2(ctr_tab, nbr_tab, center[h:], neighbor[h:])
    m1s, m1e = _edge(g1c, g1n, rbf, fcut, rsh, weights, 0)
    m2s, m2e = _edge(g2c, g2n, rbf, fcut, rsh, weights, h)
    s1, e1 = _scatter(m1s, m1e, node_invariant, node_equivariant, center[:h])
    s2, e2 = _scatter(m2s, m2e, s1, e1, center[h:])
    return s2, e2
